# baseline (speedup 1.0000x reference)
import jax
import jax.numpy as jnp
from jax.experimental import pallas as pl
from jax.experimental.pallas import tpu as pltpu

_NH = 16
_S = 32
_D = 2 * _S - 1


def _bias_body(tbl_ref, out_ref, r_scratch, m_scratch, sems):
    h = pl.program_id(0)

    tbl = tbl_ref[0]
    p = jax.lax.broadcasted_iota(jnp.int32, (64, _D), 0)
    k = jax.lax.broadcasted_iota(jnp.int32, (64, _D), 1)
    ssel = (k == 4 * (p % 16) + p // 16).astype(jnp.float32)
    trs = jnp.dot(ssel, tbl, preferred_element_type=jnp.float32)
    b = jax.lax.broadcasted_iota(jnp.int32, (_D, _S), 0)
    cj = jax.lax.broadcasted_iota(jnp.int32, (_D, _S), 1)
    for ci in range(_S):
        pc = (b == (ci - cj + (_S - 1))).astype(jnp.float32)
        pq = jnp.dot(trs, pc, preferred_element_type=jnp.float32)
        for s in range(4):
            for g in range(4):
                t = (g + s) % 4
                r_scratch[s, ci, :, g * _S:(g + 1) * _S] = (
                    pq[t * 16:(t + 1) * 16, :])

    lane = jax.lax.broadcasted_iota(jnp.int32, (_S, 15, 4 * _S), 2)
    for s in range(4):
        lo = r_scratch[s, :, 0:15, :]
        if s == 0:
            m_scratch[s, :, 0:15, :] = lo
        else:
            hi = r_scratch[s, :, 1:16, :]
            m_scratch[s, :, 0:15, :] = jnp.where(lane // _S < 4 - s, lo, hi)

    copies = []
    for r in range(_S):
        o = _S - 1 - r
        q, s = o // 4, o % 4
        cp = pltpu.make_async_copy(
            m_scratch.at[s, :, q:q + 8, :],
            out_ref.at[h, r],
            sems.at[r],
        )
        cp.start()
        copies.append(cp)
    for cp in copies:
        cp.wait()


def kernel(rel_bias, H, W):
    del H, W
    tbl = jnp.transpose(rel_bias, (2, 0, 1))[:, ::-1, :]
    out5 = pl.pallas_call(
        _bias_body,
        grid=(_NH,),
        in_specs=[pl.BlockSpec((1, _D, _D), lambda h: (h, 0, 0))],
        out_specs=pl.BlockSpec(memory_space=pl.MemorySpace.ANY),
        out_shape=jax.ShapeDtypeStruct((_NH, _S, _S, _S // 4, 4 * _S),
                                       jnp.float32),
        scratch_shapes=[
            pltpu.VMEM((4, _S, _S // 2, 4 * _S), jnp.float32),
            pltpu.VMEM((4, _S, _S // 2, 4 * _S), jnp.float32),
            pltpu.SemaphoreType.DMA((_S,)),
        ],
    )(tbl)
    return out5.reshape(_NH, _S * _S, _S * _S)
